# baseline (speedup 1.0000x reference)
import functools

import jax
import jax.numpy as jnp
from jax import lax
from jax.experimental import pallas as pl
from jax.experimental.pallas import tpu as pltpu
from jax.experimental.pallas import tpu_sc as plsc

_NT = 16
_NC = 2
_CK = 128
_JUNK = 112
_BM = 400


def _leaky(x):
    return jnp.maximum(x, 0.2 * x)


def _proj(x, W, a2):
    n, K = x.shape
    H = W.shape[1]
    NB = H // 128
    with_sd = a2 is not None

    def body(x_ref, w_ref, *rest):
        if with_sd:
            a_ref, hb_ref, sd_ref = rest
        else:
            (hb_ref,) = rest
        h = jnp.dot(x_ref[...], w_ref[...], preferred_element_type=jnp.float32)
        for b in range(NB):
            hb_ref[b, :, :] = h[:, b * 128:(b + 1) * 128]
        if with_sd:
            sd_ref[...] = jnp.dot(h, a_ref[...],
                                  preferred_element_type=jnp.float32)

    in_specs = [pl.BlockSpec((_BM, K), lambda i: (i, 0)),
                pl.BlockSpec((K, H), lambda i: (0, 0))]
    out_specs = [pl.BlockSpec((NB, _BM, 128), lambda i: (0, i, 0))]
    out_shape = [jax.ShapeDtypeStruct((NB, n, 128), jnp.float32)]
    args = [x, W]
    if with_sd:
        in_specs.append(pl.BlockSpec((H, 2), lambda i: (0, 0)))
        out_specs.append(pl.BlockSpec((_BM, 2), lambda i: (i, 0)))
        out_shape.append(jax.ShapeDtypeStruct((n, 2), jnp.float32))
        args.append(a2)
    res = pl.pallas_call(
        body, grid=(n // _BM,), in_specs=in_specs, out_specs=out_specs,
        out_shape=out_shape)(*args)
    return (res[0], res[1]) if with_sd else (res[0], None)


def _softmax_prep(sd):
    n = sd.shape[0]

    def body(sd_ref, ps_ref, mx_ref):
        asrc = sd_ref[:, 0:1]
        adst = sd_ref[:, 1:2]
        m = jnp.max(asrc)
        cdst = _leaky(m + adst)
        ps_ref[...] = jnp.exp(_leaky(asrc + adst) - cdst)
        mx_ref[...] = jnp.full((1, 128), m, jnp.float32)

    return pl.pallas_call(
        body, grid=(1,),
        in_specs=[pl.BlockSpec((n, 2), lambda i: (0, 0))],
        out_specs=[pl.BlockSpec((n, 1), lambda i: (0, 0)),
                   pl.BlockSpec((1, 128), lambda i: (0, 0))],
        out_shape=[jax.ShapeDtypeStruct((n, 1), jnp.float32),
                   jax.ShapeDtypeStruct((1, 128), jnp.float32)])(sd)


def _combine_gat(acc, hb, den, pself, bias):
    NB, n, _ = acc.shape
    H = NB * 128

    def body(acc_ref, hb_ref, den_ref, ps_ref, b_ref, o_ref):
        a = jnp.concatenate([acc_ref[b] for b in range(NB)], axis=1)
        h = jnp.concatenate([hb_ref[b] for b in range(NB)], axis=1)
        ps = ps_ref[...]
        dtot = den_ref[...] + ps
        o_ref[...] = jax.nn.relu((a + ps * h) / dtot + b_ref[...])

    return pl.pallas_call(
        body, grid=(n // _BM,),
        in_specs=[pl.BlockSpec((NB, _BM, 128), lambda i: (0, i, 0)),
                  pl.BlockSpec((NB, _BM, 128), lambda i: (0, i, 0)),
                  pl.BlockSpec((_BM, 1), lambda i: (i, 0)),
                  pl.BlockSpec((_BM, 1), lambda i: (i, 0)),
                  pl.BlockSpec((1, H), lambda i: (0, 0))],
        out_specs=pl.BlockSpec((_BM, H), lambda i: (i, 0)),
        out_shape=jax.ShapeDtypeStruct((n, H), jnp.float32),
    )(acc, hb, den, pself, bias)


def _dinv_kernel(deg):
    n = deg.shape[0]

    def body(d_ref, o_ref):
        o_ref[...] = lax.rsqrt(d_ref[...] + 1.0)

    return pl.pallas_call(
        body, grid=(1,),
        in_specs=[pl.BlockSpec((n, 1), lambda i: (0, 0))],
        out_specs=pl.BlockSpec((n, 1), lambda i: (0, 0)),
        out_shape=jax.ShapeDtypeStruct((n, 1), jnp.float32))(deg)


def _combine_gcn(acc, hb, dinv, bias):
    NB, n, _ = acc.shape
    H = NB * 128

    def body(acc_ref, hb_ref, di_ref, b_ref, o_ref):
        a = jnp.concatenate([acc_ref[b] for b in range(NB)], axis=1)
        h = jnp.concatenate([hb_ref[b] for b in range(NB)], axis=1)
        di = di_ref[...]
        o_ref[...] = a + (di * di) * h + b_ref[...]

    return pl.pallas_call(
        body, grid=(n // _BM,),
        in_specs=[pl.BlockSpec((NB, _BM, 128), lambda i: (0, i, 0)),
                  pl.BlockSpec((NB, _BM, 128), lambda i: (0, i, 0)),
                  pl.BlockSpec((_BM, 1), lambda i: (i, 0)),
                  pl.BlockSpec((1, H), lambda i: (0, 0))],
        out_specs=pl.BlockSpec((_BM, H), lambda i: (i, 0)),
        out_shape=jax.ShapeDtypeStruct((n, H), jnp.float32),
    )(acc, hb, dinv, bias)


def _combine_proj(acc, hb, den, pself, bias, W, a2, deg):
    NB, n, _ = acc.shape
    H = NB * 128
    Hn = W.shape[1]
    NBn = Hn // 128
    with_sd = a2 is not None
    with_dinv = deg is not None

    def body(acc_ref, hb_ref, den_ref, ps_ref, b_ref, w_ref, *rest):
        i = 0
        a_ref = d_ref = None
        if with_sd:
            a_ref = rest[i]; i += 1
        if with_dinv:
            d_ref = rest[i]; i += 1
        hbn_ref = rest[i]; i += 1
        sd_ref = di_ref = None
        if with_sd:
            sd_ref = rest[i]; i += 1
        if with_dinv:
            di_ref = rest[i]; i += 1
        a = jnp.concatenate([acc_ref[b] for b in range(NB)], axis=1)
        h = jnp.concatenate([hb_ref[b] for b in range(NB)], axis=1)
        ps = ps_ref[...]
        dtot = den_ref[...] + ps
        X = jax.nn.relu((a + ps * h) / dtot + b_ref[...])
        hn = jnp.dot(X, w_ref[...], preferred_element_type=jnp.float32)
        for b in range(NBn):
            hbn_ref[b, :, :] = hn[:, b * 128:(b + 1) * 128]
        if with_sd:
            sd_ref[...] = jnp.dot(hn, a_ref[...],
                                  preferred_element_type=jnp.float32)
        if with_dinv:
            di_ref[...] = lax.rsqrt(d_ref[...] + 1.0)

    in_specs = [pl.BlockSpec((NB, _BM, 128), lambda i: (0, i, 0)),
                pl.BlockSpec((NB, _BM, 128), lambda i: (0, i, 0)),
                pl.BlockSpec((_BM, 1), lambda i: (i, 0)),
                pl.BlockSpec((_BM, 1), lambda i: (i, 0)),
                pl.BlockSpec((1, H), lambda i: (0, 0)),
                pl.BlockSpec((H, Hn), lambda i: (0, 0))]
    args = [acc, hb, den, pself, bias, W]
    out_specs = [pl.BlockSpec((NBn, _BM, 128), lambda i: (0, i, 0))]
    out_shape = [jax.ShapeDtypeStruct((NBn, n, 128), jnp.float32)]
    if with_sd:
        in_specs.append(pl.BlockSpec((Hn, 2), lambda i: (0, 0)))
        args.append(a2)
        out_specs.append(pl.BlockSpec((_BM, 2), lambda i: (i, 0)))
        out_shape.append(jax.ShapeDtypeStruct((n, 2), jnp.float32))
    if with_dinv:
        in_specs.append(pl.BlockSpec((_BM, 1), lambda i: (i, 0)))
        args.append(deg)
        out_specs.append(pl.BlockSpec((_BM, 1), lambda i: (i, 0)))
        out_shape.append(jax.ShapeDtypeStruct((n, 1), jnp.float32))
    res = pl.pallas_call(
        body, grid=(n // _BM,), in_specs=in_specs, out_specs=out_specs,
        out_shape=out_shape)(*args)
    hbn = res[0]
    sd = res[1] if with_sd else None
    dinv = res[-1] if with_dinv else None
    return hbn, sd, dinv


def _row_chunks(total, step):
    out, off = [], 0
    while off < total:
        out.append((off, min(step, total - off)))
        off += step
    return out


def _sc_edge(hb, vals, mx, src3, dst3, n, mode, nch):
    NBn = hb.shape[0]
    NB = NBn // n
    NPB = NB // _NC
    NP = n + _JUNK
    RPT = NP // _NT
    gat = mode in ("gat", "gat_deg")
    with_deg = mode == "gat_deg"
    nv = len(vals)
    NPAIR = nch // 2

    mesh = plsc.VectorSubcoreMesh(core_axis_name="c", subcore_axis_name="s")

    out_type = [jax.ShapeDtypeStruct((NB * NP, 128), jnp.float32)]
    if gat:
        out_type.append(jax.ShapeDtypeStruct((NP,), jnp.float32))
    if with_deg:
        out_type.append(jax.ShapeDtypeStruct((NP,), jnp.float32))

    def vm(shape, dt):
        return pltpu.VMEM(shape, dt)

    scratch = (
        [vm((128,), jnp.int32) for _ in range(2)] +
        [vm((1, 128), jnp.int32) for _ in range(2)] +
        [vm((128,), jnp.int32) for _ in range(2)] +
        [vm((128,), jnp.float32) for _ in range(2)] +
        [vm((128,), jnp.float32) for _ in range(2)] +
        [vm((1, 128), jnp.float32) for _ in range(2)] +
        [vm((nch if NPB > 1 else 1, _CK), jnp.float32)] +
        [vm((128, 128), jnp.float32) for _ in range(2)] +
        [vm((1, 128), jnp.float32),
         vm((128,), jnp.float32),
         vm((128,), jnp.float32)] +
        [pltpu.SemaphoreType.DMA for _ in range(6)] +
        [pltpu.VMEM_SHARED((NP,), jnp.float32) for _ in range(nv)] +
        [pltpu.VMEM_SHARED((NP, 128), jnp.float32),
         pltpu.VMEM_SHARED((NP,), jnp.float32)])

    def body(hb_ref, *refs):
        i = 0
        val_refs = refs[i:i + nv]; i += nv
        mx_ref = None
        if gat:
            mx_ref = refs[i]; i += 1
        src_ref, dst_ref = refs[i:i + 2]; i += 2
        acc_ref = refs[i]; i += 1
        den_ref = deg_ref = None
        if gat:
            den_ref = refs[i]; i += 1
        if with_deg:
            deg_ref = refs[i]; i += 1
        src_c = refs[i:i + 2]; i += 2
        dst_c = refs[i:i + 2]; i += 2
        srco_c = refs[i:i + 2]; i += 2
        g1 = refs[i:i + 2]; i += 2
        g2 = refs[i:i + 2]; i += 2
        p_c = refs[i:i + 2]; i += 2
        p_v = refs[i]; i += 1
        rowbuf = refs[i:i + 2]; i += 2
        ones_c, mx_c, bounce_v = refs[i:i + 3]; i += 3
        sem_row = refs[i:i + 2]; i += 2
        sem_g = refs[i:i + 2]; i += 2
        sem_sc = refs[i:i + 2]; i += 2
        val_sp = refs[i:i + nv]; i += nv
        acc_sp, den_sp = refs[i:i + 2]

        c = lax.axis_index("c")
        s = lax.axis_index("s")
        row0 = s * RPT
        zeros16 = jnp.zeros((16,), jnp.float32)
        rslices = _row_chunks(RPT, 128)

        for vr, vs in zip(val_refs, val_sp):
            for off, sz in rslices:
                pltpu.sync_copy(vr.at[pl.ds(row0 + off, sz)],
                                bounce_v.at[pl.ds(0, sz)])
                pltpu.sync_copy(bounce_v.at[pl.ds(0, sz)],
                                vs.at[pl.ds(row0 + off, sz)])
        if gat:
            pltpu.sync_copy(mx_ref.at[0], mx_c)
        for u in range(8):
            ones_c[0, pl.ds(16 * u, 16)] = jnp.full((16,), 1.0, jnp.float32)

        def zero_rowbuf0(_j, _):
            for u in range(8):
                rowbuf[0][_j, pl.ds(16 * u, 16)] = zeros16
            return 0

        lax.fori_loop(0, 128, zero_rowbuf0, 0)
        if gat:
            for off, sz in rslices:
                pltpu.sync_copy(rowbuf[0].at[0, pl.ds(0, sz)],
                                den_sp.at[pl.ds(row0 + off, sz)])
        plsc.subcore_barrier()

        def idx_load(ch, sl):
            pltpu.sync_copy(src_ref.at[s, ch], src_c[sl])
            pltpu.sync_copy(dst_ref.at[s, ch], dst_c[sl].at[0])

        def gather_descs(sl, with_vals):
            vsl = val_sp[1] if gat else val_sp[0]
            ds = [pltpu.make_async_copy(hb_ref.at[srco_c[sl]], rowbuf[sl],
                                        sem_row[sl])]
            if with_vals:
                ds.append(pltpu.make_async_copy(val_sp[0].at[src_c[sl]],
                                                g1[sl], sem_g[sl]))
                ds.append(pltpu.make_async_copy(vsl.at[dst_c[sl].at[0]],
                                                g2[sl], sem_g[sl]))
            return ds

        def issue_gathers(sl, with_vals):
            for d in gather_descs(sl, with_vals):
                d.start()

        def wait_gathers(sl, with_vals):
            for d in gather_descs(sl, with_vals):
                d.wait()

        def drain_sc(sl):
            pltpu.make_async_copy(rowbuf[sl],
                                  acc_sp.at[dst_c[sl].at[0]],
                                  sem_sc[sl]).wait()

        def compute_srco(sl, boffn):
            for u in range(8):
                srco_c[sl][pl.ds(16 * u, 16)] = (
                    src_c[sl][pl.ds(16 * u, 16)] + boffn)

        def process(sl, den_pass, ch, first_pass):
            wait_gathers(sl, first_pass)
            if first_pass:
                for g8 in range(8):
                    fsl = pl.ds(16 * g8, 16)
                    if gat:
                        t = g1[sl][fsl] + g2[sl][fsl]
                        u_ = mx_c[pl.ds(0, 16)] + g2[sl][fsl]
                        p = jnp.exp(_leaky(t) - _leaky(u_))
                    else:
                        p = g1[sl][fsl] * g2[sl][fsl]
                    p_c[sl][0, fsl] = p
                    if NPB > 1:
                        p_v[ch, fsl] = p
            if den_pass:
                @pl.when(c == 0)
                def _():
                    pltpu.sync_copy(p_c[sl].at[0],
                                    den_sp.at[dst_c[sl].at[0]], add=True)
                if with_deg:
                    @pl.when(c == 1)
                    def _():
                        pltpu.sync_copy(ones_c.at[0],
                                        den_sp.at[dst_c[sl].at[0]],
                                        add=True)
            def sgrp(g, _):
                if first_pass:
                    pv16 = p_c[sl][0, pl.ds(16 * g, 16)]
                else:
                    pv16 = p_v[ch, pl.ds(16 * g, 16)]
                for l in range(16):
                    j = 16 * g + l
                    pj = pv16[l]
                    for u in range(8):
                        rowbuf[sl][j, pl.ds(16 * u, 16)] = (
                            rowbuf[sl][j, pl.ds(16 * u, 16)] * pj)
                return 0
            lax.fori_loop(0, 8, sgrp, 0, unroll=4)
            pltpu.make_async_copy(rowbuf[sl],
                                  acc_sp.at[dst_c[sl].at[0]],
                                  sem_sc[sl]).start(add=True)

        for bi in range(NPB):
            blk = c * NPB + bi
            boffn = blk * n
            den_pass = gat and bi == 0

            if bi > 0:
                def zero_rb(_j, _):
                    for u in range(8):
                        rowbuf[0][_j, pl.ds(16 * u, 16)] = zeros16
                    return 0
                lax.fori_loop(0, 128, zero_rb, 0)
            for off, sz in rslices:
                pltpu.sync_copy(rowbuf[0].at[pl.ds(0, sz)],
                                acc_sp.at[pl.ds(row0 + off, sz)])
            plsc.subcore_barrier()

            fp = bi == 0
            idx_load(0, 0)
            compute_srco(0, boffn)
            issue_gathers(0, fp)
            idx_load(nch, 1)
            pltpu.make_async_copy(rowbuf[1],
                                  acc_sp.at[dst_c[1].at[0]],
                                  sem_sc[1]).start(add=True)

            def pair_body(g, _):
                a = 2 * g
                drain_sc(1)
                idx_load(a + 1, 1)
                compute_srco(1, boffn)
                issue_gathers(1, fp)
                process(0, den_pass, a, fp)
                process(1, den_pass, a + 1, fp)
                drain_sc(0)
                idx_load(a + 2, 0)
                compute_srco(0, boffn)
                issue_gathers(0, fp)
                return 0
            lax.fori_loop(0, NPAIR, pair_body, 0)

            drain_sc(1)
            wait_gathers(0, fp)
            plsc.subcore_barrier()

            if den_pass:
                for off, sz in rslices:
                    @pl.when(c == 0)
                    def _(off=off, sz=sz):
                        pltpu.sync_copy(den_sp.at[pl.ds(row0 + off, sz)],
                                        bounce_v.at[pl.ds(0, sz)])
                        pltpu.sync_copy(bounce_v.at[pl.ds(0, sz)],
                                        den_ref.at[pl.ds(row0 + off, sz)])
                if with_deg:
                    for off, sz in rslices:
                        @pl.when(c == 1)
                        def _(off=off, sz=sz):
                            pltpu.sync_copy(den_sp.at[pl.ds(row0 + off, sz)],
                                            bounce_v.at[pl.ds(0, sz)])
                            pltpu.sync_copy(bounce_v.at[pl.ds(0, sz)],
                                            deg_ref.at[pl.ds(row0 + off, sz)])
            boffp = blk * NP
            for off, sz in rslices:
                pltpu.sync_copy(acc_sp.at[pl.ds(row0 + off, sz)],
                                rowbuf[1].at[pl.ds(0, sz)])
                pltpu.sync_copy(rowbuf[1].at[pl.ds(0, sz)],
                                acc_ref.at[pl.ds(boffp + row0 + off, sz)])
            plsc.subcore_barrier()

    fn = pl.kernel(body, out_type=out_type, mesh=mesh,
                   scratch_types=scratch,
                   compiler_params=pltpu.CompilerParams(
                       needs_layout_passes=False))
    args = [hb] + list(vals)
    if gat:
        args.append(mx)
    args += [src3, dst3]
    res = fn(*args)
    if not isinstance(res, (tuple, list)):
        res = (res,)
    accf = res[0].reshape(NB, NP, 128)
    den = res[1] if gat else None
    deg = res[2] if with_deg else None
    return accf, den, deg


def _sc_gat(hb, sd, src3, dst3, n, nch, with_deg):
    pself, mx = _softmax_prep(sd)
    asrc = sd[:, 0]
    adst = sd[:, 1]
    NB = hb.shape[0]
    hflat = hb.reshape(NB * n, 128)
    pad = ((0, _JUNK),)
    acc, den, deg = _sc_edge(
        hflat, (jnp.pad(asrc, pad), jnp.pad(adst, pad)), mx,
        src3, dst3, n, "gat_deg" if with_deg else "gat", nch)
    return acc[:, :n, :], den[:n].reshape(n, 1), pself, deg


def kernel(x, edge_index, W1, a_src1, a_dst1, b1, W2, a_src2, a_dst2, b2,
           W3, a_src3, a_dst3, b3, W4, b4):
    n = x.shape[0]
    e = edge_index.shape[1]
    src = edge_index[0].astype(jnp.int32)
    dst = edge_index[1].astype(jnp.int32)

    ept = -(-e // _NT)
    nch = -(-ept // _CK)
    nch += nch % 2
    ncha = nch + 2
    padlen = _NT * nch * _CK - e
    ar = jnp.arange(padlen, dtype=jnp.int32)
    src_p = jnp.concatenate([src, ar % n]).reshape(_NT, nch, _CK)
    dst_p = jnp.concatenate([dst, n + (ar % _JUNK)]).reshape(_NT, nch, _CK)
    arj = jnp.arange(_NT * 2 * _CK, dtype=jnp.int32)
    srcj = (arj % n).reshape(_NT, 2, _CK)
    dstj = (n + (arj % _JUNK)).reshape(_NT, 2, _CK)
    src3 = jnp.concatenate([src_p, srcj], axis=1)
    dst3 = jnp.concatenate([dst_p, dstj], axis=1)

    a2_1 = jnp.stack([a_src1, a_dst1], axis=1)
    a2_2 = jnp.stack([a_src2, a_dst2], axis=1)
    a2_3 = jnp.stack([a_src3, a_dst3], axis=1)

    hb1, sd1 = _proj(x, W1, a2_1)
    acc1, den1, ps1, deg = _sc_gat(hb1, sd1, src3, dst3, n, nch, True)
    hb2, sd2, _ = _combine_proj(acc1, hb1, den1, ps1, b1.reshape(1, -1),
                                W2, a2_2, None)
    acc2, den2, ps2, _ = _sc_gat(hb2, sd2, src3, dst3, n, nch, False)
    hb3, sd3, _ = _combine_proj(acc2, hb2, den2, ps2, b2.reshape(1, -1),
                                W3, a2_3, None)
    acc3, den3, ps3, _ = _sc_gat(hb3, sd3, src3, dst3, n, nch, False)
    hb4, _, dinv = _combine_proj(acc3, hb3, den3, ps3, b3.reshape(1, -1),
                                 W4, None, deg[:n].reshape(n, 1))

    NB4 = hb4.shape[0]
    acc4, _, _ = _sc_edge(hb4.reshape(NB4 * n, 128),
                          (jnp.pad(dinv[:, 0], ((0, _JUNK),)),), None,
                          src3, dst3, n, "gcn", nch)
    z = _combine_gcn(acc4[:, :n, :], hb4, dinv, b4.reshape(1, -1))
    return z


# speedup vs baseline: 1.0375x; 1.0375x over previous
import functools

import jax
import jax.numpy as jnp
from jax import lax
from jax.experimental import pallas as pl
from jax.experimental.pallas import tpu as pltpu
from jax.experimental.pallas import tpu_sc as plsc

_NT = 16
_NC = 2
_CK = 128
_JUNK = 112
_BM = 400


def _leaky(x):
    return jnp.maximum(x, 0.2 * x)


def _proj(x, W, a2):
    n, K = x.shape
    H = W.shape[1]
    NB = H // 128
    with_sd = a2 is not None

    def body(x_ref, w_ref, *rest):
        if with_sd:
            a_ref, hb_ref, sd_ref, mx_ref = rest
        else:
            (hb_ref,) = rest
        h = jnp.dot(x_ref[...], w_ref[...], preferred_element_type=jnp.float32)
        for b in range(NB):
            hb_ref[b, :, :] = h[:, b * 128:(b + 1) * 128]
        if with_sd:
            sdb = jnp.dot(h, a_ref[...], preferred_element_type=jnp.float32)
            sd_ref[...] = sdb
            i = pl.program_id(0)

            @pl.when(i == 0)
            def _():
                mx_ref[...] = jnp.full((1, 128), -jnp.inf, jnp.float32)
            mx_ref[...] = jnp.maximum(mx_ref[...], jnp.max(sdb[:, 0:1]))

    in_specs = [pl.BlockSpec((_BM, K), lambda i: (i, 0)),
                pl.BlockSpec((K, H), lambda i: (0, 0))]
    out_specs = [pl.BlockSpec((NB, _BM, 128), lambda i: (0, i, 0))]
    out_shape = [jax.ShapeDtypeStruct((NB, n, 128), jnp.float32)]
    args = [x, W]
    if with_sd:
        in_specs.append(pl.BlockSpec((H, 2), lambda i: (0, 0)))
        out_specs.append(pl.BlockSpec((_BM, 2), lambda i: (i, 0)))
        out_shape.append(jax.ShapeDtypeStruct((n, 2), jnp.float32))
        out_specs.append(pl.BlockSpec((1, 128), lambda i: (0, 0)))
        out_shape.append(jax.ShapeDtypeStruct((1, 128), jnp.float32))
        args.append(a2)
    res = pl.pallas_call(
        body, grid=(n // _BM,), in_specs=in_specs, out_specs=out_specs,
        out_shape=out_shape)(*args)
    return (res[0], res[1], res[2]) if with_sd else (res[0], None, None)


def _combine_gcn(acc, hb, dinv, bias):
    NB, n, _ = acc.shape
    H = NB * 128

    def body(acc_ref, hb_ref, di_ref, b_ref, o_ref):
        a = jnp.concatenate([acc_ref[b] for b in range(NB)], axis=1)
        h = jnp.concatenate([hb_ref[b] for b in range(NB)], axis=1)
        di = di_ref[...]
        o_ref[...] = a + (di * di) * h + b_ref[...]

    return pl.pallas_call(
        body, grid=(n // _BM,),
        in_specs=[pl.BlockSpec((NB, _BM, 128), lambda i: (0, i, 0)),
                  pl.BlockSpec((NB, _BM, 128), lambda i: (0, i, 0)),
                  pl.BlockSpec((_BM, 1), lambda i: (i, 0)),
                  pl.BlockSpec((1, H), lambda i: (0, 0))],
        out_specs=pl.BlockSpec((_BM, H), lambda i: (i, 0)),
        out_shape=jax.ShapeDtypeStruct((n, H), jnp.float32),
    )(acc, hb, dinv, bias)


def _combine_proj(acc, hb, den, sd_prev, mx_prev, bias, W, a2, deg):
    NB, n, _ = acc.shape
    H = NB * 128
    Hn = W.shape[1]
    NBn = Hn // 128
    with_sd = a2 is not None
    with_dinv = deg is not None

    def body(acc_ref, hb_ref, den_ref, sp_ref, mp_ref, b_ref, w_ref, *rest):
        i = 0
        a_ref = d_ref = None
        if with_sd:
            a_ref = rest[i]; i += 1
        if with_dinv:
            d_ref = rest[i]; i += 1
        hbn_ref = rest[i]; i += 1
        sd_ref = mx_ref = di_ref = None
        if with_sd:
            sd_ref = rest[i]; i += 1
            mx_ref = rest[i]; i += 1
        if with_dinv:
            di_ref = rest[i]; i += 1
        a = jnp.concatenate([acc_ref[b] for b in range(NB)], axis=1)
        h = jnp.concatenate([hb_ref[b] for b in range(NB)], axis=1)
        asrc = sp_ref[:, 0:1]
        adst = sp_ref[:, 1:2]
        m = mp_ref[0:1, 0:1]
        ps = jnp.exp(_leaky(asrc + adst) - _leaky(m + adst))
        dtot = den_ref[...] + ps
        X = jax.nn.relu((a + ps * h) / dtot + b_ref[...])
        hn = jnp.dot(X, w_ref[...], preferred_element_type=jnp.float32)
        for b in range(NBn):
            hbn_ref[b, :, :] = hn[:, b * 128:(b + 1) * 128]
        if with_sd:
            sdb = jnp.dot(hn, a_ref[...], preferred_element_type=jnp.float32)
            sd_ref[...] = sdb
            gi = pl.program_id(0)

            @pl.when(gi == 0)
            def _():
                mx_ref[...] = jnp.full((1, 128), -jnp.inf, jnp.float32)
            mx_ref[...] = jnp.maximum(mx_ref[...], jnp.max(sdb[:, 0:1]))
        if with_dinv:
            di_ref[...] = lax.rsqrt(d_ref[...] + 1.0)

    in_specs = [pl.BlockSpec((NB, _BM, 128), lambda i: (0, i, 0)),
                pl.BlockSpec((NB, _BM, 128), lambda i: (0, i, 0)),
                pl.BlockSpec((_BM, 1), lambda i: (i, 0)),
                pl.BlockSpec((_BM, 2), lambda i: (i, 0)),
                pl.BlockSpec((1, 128), lambda i: (0, 0)),
                pl.BlockSpec((1, H), lambda i: (0, 0)),
                pl.BlockSpec((H, Hn), lambda i: (0, 0))]
    args = [acc, hb, den, sd_prev, mx_prev, bias, W]
    out_specs = [pl.BlockSpec((NBn, _BM, 128), lambda i: (0, i, 0))]
    out_shape = [jax.ShapeDtypeStruct((NBn, n, 128), jnp.float32)]
    if with_sd:
        in_specs.append(pl.BlockSpec((Hn, 2), lambda i: (0, 0)))
        args.append(a2)
        out_specs.append(pl.BlockSpec((_BM, 2), lambda i: (i, 0)))
        out_shape.append(jax.ShapeDtypeStruct((n, 2), jnp.float32))
        out_specs.append(pl.BlockSpec((1, 128), lambda i: (0, 0)))
        out_shape.append(jax.ShapeDtypeStruct((1, 128), jnp.float32))
    if with_dinv:
        in_specs.append(pl.BlockSpec((_BM, 1), lambda i: (i, 0)))
        args.append(deg)
        out_specs.append(pl.BlockSpec((_BM, 1), lambda i: (i, 0)))
        out_shape.append(jax.ShapeDtypeStruct((n, 1), jnp.float32))
    res = pl.pallas_call(
        body, grid=(n // _BM,), in_specs=in_specs, out_specs=out_specs,
        out_shape=out_shape)(*args)
    hbn = res[0]
    sd = res[1] if with_sd else None
    mx = res[2] if with_sd else None
    dinv = res[-1] if with_dinv else None
    return hbn, sd, mx, dinv


def _row_chunks(total, step):
    out, off = [], 0
    while off < total:
        out.append((off, min(step, total - off)))
        off += step
    return out


def _sc_edge(hb, vals, mx, src3, dst3, n, mode, nch):
    NBn = hb.shape[0]
    NB = NBn // n
    NPB = NB // _NC
    NP = n + _JUNK
    RPT = NP // _NT
    gat = mode in ("gat", "gat_deg")
    with_deg = mode == "gat_deg"
    nv = len(vals)
    NPAIR = nch // 2

    mesh = plsc.VectorSubcoreMesh(core_axis_name="c", subcore_axis_name="s")

    out_type = [jax.ShapeDtypeStruct((NB * NP, 128), jnp.float32)]
    if gat:
        out_type.append(jax.ShapeDtypeStruct((NP,), jnp.float32))
    if with_deg:
        out_type.append(jax.ShapeDtypeStruct((NP,), jnp.float32))

    def vm(shape, dt):
        return pltpu.VMEM(shape, dt)

    scratch = (
        [vm((128,), jnp.int32) for _ in range(2)] +
        [vm((1, 128), jnp.int32) for _ in range(2)] +
        [vm((128,), jnp.int32) for _ in range(2)] +
        [vm((128,), jnp.float32) for _ in range(2)] +
        [vm((128,), jnp.float32) for _ in range(2)] +
        [vm((1, 128), jnp.float32) for _ in range(2)] +
        [vm((nch if NPB > 1 else 1, _CK), jnp.float32)] +
        [vm((128, 128), jnp.float32) for _ in range(2)] +
        [vm((1, 128), jnp.float32),
         vm((128,), jnp.float32),
         vm((128,), jnp.float32)] +
        [pltpu.SemaphoreType.DMA for _ in range(6)] +
        [pltpu.VMEM_SHARED((NP,), jnp.float32) for _ in range(nv)] +
        [pltpu.VMEM_SHARED((NP, 128), jnp.float32),
         pltpu.VMEM_SHARED((NP,), jnp.float32)])

    def body(hb_ref, *refs):
        i = 0
        val_refs = refs[i:i + nv]; i += nv
        mx_ref = None
        if gat:
            mx_ref = refs[i]; i += 1
        src_ref, dst_ref = refs[i:i + 2]; i += 2
        acc_ref = refs[i]; i += 1
        den_ref = deg_ref = None
        if gat:
            den_ref = refs[i]; i += 1
        if with_deg:
            deg_ref = refs[i]; i += 1
        src_c = refs[i:i + 2]; i += 2
        dst_c = refs[i:i + 2]; i += 2
        srco_c = refs[i:i + 2]; i += 2
        g1 = refs[i:i + 2]; i += 2
        g2 = refs[i:i + 2]; i += 2
        p_c = refs[i:i + 2]; i += 2
        p_v = refs[i]; i += 1
        rowbuf = refs[i:i + 2]; i += 2
        ones_c, mx_c, bounce_v = refs[i:i + 3]; i += 3
        sem_row = refs[i:i + 2]; i += 2
        sem_g = refs[i:i + 2]; i += 2
        sem_sc = refs[i:i + 2]; i += 2
        val_sp = refs[i:i + nv]; i += nv
        acc_sp, den_sp = refs[i:i + 2]

        c = lax.axis_index("c")
        s = lax.axis_index("s")
        row0 = s * RPT
        zeros16 = jnp.zeros((16,), jnp.float32)
        rslices = _row_chunks(RPT, 128)

        for vr, vs in zip(val_refs, val_sp):
            for off, sz in rslices:
                pltpu.sync_copy(vr.at[pl.ds(row0 + off, sz)],
                                bounce_v.at[pl.ds(0, sz)])
                pltpu.sync_copy(bounce_v.at[pl.ds(0, sz)],
                                vs.at[pl.ds(row0 + off, sz)])
        if gat:
            pltpu.sync_copy(mx_ref.at[0], mx_c)
        for u in range(8):
            ones_c[0, pl.ds(16 * u, 16)] = jnp.full((16,), 1.0, jnp.float32)

        def zero_rowbuf0(_j, _):
            for u in range(8):
                rowbuf[0][_j, pl.ds(16 * u, 16)] = zeros16
            return 0

        lax.fori_loop(0, 128, zero_rowbuf0, 0)
        if gat:
            for off, sz in rslices:
                pltpu.sync_copy(rowbuf[0].at[0, pl.ds(0, sz)],
                                den_sp.at[pl.ds(row0 + off, sz)])
        plsc.subcore_barrier()

        def idx_load(ch, sl):
            pltpu.sync_copy(src_ref.at[s, ch], src_c[sl])
            pltpu.sync_copy(dst_ref.at[s, ch], dst_c[sl].at[0])

        def gather_descs(sl, with_vals):
            vsl = val_sp[1] if gat else val_sp[0]
            ds = [pltpu.make_async_copy(hb_ref.at[srco_c[sl]], rowbuf[sl],
                                        sem_row[sl])]
            if with_vals:
                ds.append(pltpu.make_async_copy(val_sp[0].at[src_c[sl]],
                                                g1[sl], sem_g[sl]))
                ds.append(pltpu.make_async_copy(vsl.at[dst_c[sl].at[0]],
                                                g2[sl], sem_g[sl]))
            return ds

        def issue_gathers(sl, with_vals):
            for d in gather_descs(sl, with_vals):
                d.start()

        def wait_gathers(sl, with_vals):
            for d in gather_descs(sl, with_vals):
                d.wait()

        def drain_sc(sl):
            pltpu.make_async_copy(rowbuf[sl],
                                  acc_sp.at[dst_c[sl].at[0]],
                                  sem_sc[sl]).wait()

        def compute_srco(sl, boffn):
            for u in range(8):
                srco_c[sl][pl.ds(16 * u, 16)] = (
                    src_c[sl][pl.ds(16 * u, 16)] + boffn)

        def process(sl, den_pass, ch, first_pass):
            wait_gathers(sl, first_pass)
            if first_pass:
                for g8 in range(8):
                    fsl = pl.ds(16 * g8, 16)
                    if gat:
                        t = g1[sl][fsl] + g2[sl][fsl]
                        u_ = mx_c[pl.ds(0, 16)] + g2[sl][fsl]
                        p = jnp.exp(_leaky(t) - _leaky(u_))
                    else:
                        p = g1[sl][fsl] * g2[sl][fsl]
                    p_c[sl][0, fsl] = p
                    if NPB > 1:
                        p_v[ch, fsl] = p
            if den_pass:
                @pl.when(c == 0)
                def _():
                    pltpu.sync_copy(p_c[sl].at[0],
                                    den_sp.at[dst_c[sl].at[0]], add=True)
                if with_deg:
                    @pl.when(c == 1)
                    def _():
                        pltpu.sync_copy(ones_c.at[0],
                                        den_sp.at[dst_c[sl].at[0]],
                                        add=True)
            def sgrp(g, _):
                if first_pass:
                    pv16 = p_c[sl][0, pl.ds(16 * g, 16)]
                else:
                    pv16 = p_v[ch, pl.ds(16 * g, 16)]
                for l in range(16):
                    j = 16 * g + l
                    pj = pv16[l]
                    for u in range(8):
                        rowbuf[sl][j, pl.ds(16 * u, 16)] = (
                            rowbuf[sl][j, pl.ds(16 * u, 16)] * pj)
                return 0
            lax.fori_loop(0, 8, sgrp, 0, unroll=2)
            pltpu.make_async_copy(rowbuf[sl],
                                  acc_sp.at[dst_c[sl].at[0]],
                                  sem_sc[sl]).start(add=True)

        for bi in range(NPB):
            blk = c * NPB + bi
            boffn = blk * n
            den_pass = gat and bi == 0

            if bi > 0:
                def zero_rb(_j, _):
                    for u in range(8):
                        rowbuf[0][_j, pl.ds(16 * u, 16)] = zeros16
                    return 0
                lax.fori_loop(0, 128, zero_rb, 0)
            for off, sz in rslices:
                pltpu.sync_copy(rowbuf[0].at[pl.ds(0, sz)],
                                acc_sp.at[pl.ds(row0 + off, sz)])
            plsc.subcore_barrier()

            fp = bi == 0
            idx_load(0, 0)
            compute_srco(0, boffn)
            issue_gathers(0, fp)
            idx_load(nch, 1)
            pltpu.make_async_copy(rowbuf[1],
                                  acc_sp.at[dst_c[1].at[0]],
                                  sem_sc[1]).start(add=True)

            def pair_body(g, _):
                a = 2 * g
                drain_sc(1)
                idx_load(a + 1, 1)
                compute_srco(1, boffn)
                issue_gathers(1, fp)
                process(0, den_pass, a, fp)
                process(1, den_pass, a + 1, fp)
                drain_sc(0)
                idx_load(a + 2, 0)
                compute_srco(0, boffn)
                issue_gathers(0, fp)
                return 0
            lax.fori_loop(0, NPAIR, pair_body, 0)

            drain_sc(1)
            wait_gathers(0, fp)
            plsc.subcore_barrier()

            if den_pass:
                for off, sz in rslices:
                    @pl.when(c == 0)
                    def _(off=off, sz=sz):
                        pltpu.sync_copy(den_sp.at[pl.ds(row0 + off, sz)],
                                        bounce_v.at[pl.ds(0, sz)])
                        pltpu.sync_copy(bounce_v.at[pl.ds(0, sz)],
                                        den_ref.at[pl.ds(row0 + off, sz)])
                if with_deg:
                    for off, sz in rslices:
                        @pl.when(c == 1)
                        def _(off=off, sz=sz):
                            pltpu.sync_copy(den_sp.at[pl.ds(row0 + off, sz)],
                                            bounce_v.at[pl.ds(0, sz)])
                            pltpu.sync_copy(bounce_v.at[pl.ds(0, sz)],
                                            deg_ref.at[pl.ds(row0 + off, sz)])
            boffp = blk * NP
            for off, sz in rslices:
                pltpu.sync_copy(acc_sp.at[pl.ds(row0 + off, sz)],
                                rowbuf[1].at[pl.ds(0, sz)])
                pltpu.sync_copy(rowbuf[1].at[pl.ds(0, sz)],
                                acc_ref.at[pl.ds(boffp + row0 + off, sz)])
            plsc.subcore_barrier()

    fn = pl.kernel(body, out_type=out_type, mesh=mesh,
                   scratch_types=scratch,
                   compiler_params=pltpu.CompilerParams(
                       needs_layout_passes=False))
    args = [hb] + list(vals)
    if gat:
        args.append(mx)
    args += [src3, dst3]
    res = fn(*args)
    if not isinstance(res, (tuple, list)):
        res = (res,)
    accf = res[0].reshape(NB, NP, 128)
    den = res[1] if gat else None
    deg = res[2] if with_deg else None
    return accf, den, deg


def _sc_gat(hb, sd, mx, src3, dst3, n, nch, with_deg):
    asrc = sd[:, 0]
    adst = sd[:, 1]
    NB = hb.shape[0]
    hflat = hb.reshape(NB * n, 128)
    pad = ((0, _JUNK),)
    acc, den, deg = _sc_edge(
        hflat, (jnp.pad(asrc, pad), jnp.pad(adst, pad)), mx,
        src3, dst3, n, "gat_deg" if with_deg else "gat", nch)
    return acc[:, :n, :], den[:n].reshape(n, 1), deg


def kernel(x, edge_index, W1, a_src1, a_dst1, b1, W2, a_src2, a_dst2, b2,
           W3, a_src3, a_dst3, b3, W4, b4):
    n = x.shape[0]
    e = edge_index.shape[1]
    src = edge_index[0].astype(jnp.int32)
    dst = edge_index[1].astype(jnp.int32)

    ept = -(-e // _NT)
    nch = -(-ept // _CK)
    nch += nch % 2
    ncha = nch + 2
    padlen = _NT * nch * _CK - e
    ar = jnp.arange(padlen, dtype=jnp.int32)
    src_p = jnp.concatenate([src, ar % n]).reshape(_NT, nch, _CK)
    dst_p = jnp.concatenate([dst, n + (ar % _JUNK)]).reshape(_NT, nch, _CK)
    arj = jnp.arange(_NT * 2 * _CK, dtype=jnp.int32)
    srcj = (arj % n).reshape(_NT, 2, _CK)
    dstj = (n + (arj % _JUNK)).reshape(_NT, 2, _CK)
    src3 = jnp.concatenate([src_p, srcj], axis=1)
    dst3 = jnp.concatenate([dst_p, dstj], axis=1)

    a2_1 = jnp.stack([a_src1, a_dst1], axis=1)
    a2_2 = jnp.stack([a_src2, a_dst2], axis=1)
    a2_3 = jnp.stack([a_src3, a_dst3], axis=1)

    hb1, sd1, mx1 = _proj(x, W1, a2_1)
    acc1, den1, deg = _sc_gat(hb1, sd1, mx1, src3, dst3, n, nch, True)
    hb2, sd2, mx2, _ = _combine_proj(acc1, hb1, den1, sd1, mx1,
                                     b1.reshape(1, -1), W2, a2_2, None)
    acc2, den2, _ = _sc_gat(hb2, sd2, mx2, src3, dst3, n, nch, False)
    hb3, sd3, mx3, _ = _combine_proj(acc2, hb2, den2, sd2, mx2,
                                     b2.reshape(1, -1), W3, a2_3, None)
    acc3, den3, _ = _sc_gat(hb3, sd3, mx3, src3, dst3, n, nch, False)
    hb4, _, _, dinv = _combine_proj(acc3, hb3, den3, sd3, mx3,
                                    b3.reshape(1, -1), W4, None,
                                    deg[:n].reshape(n, 1))

    NB4 = hb4.shape[0]
    acc4, _, _ = _sc_edge(hb4.reshape(NB4 * n, 128),
                          (jnp.pad(dinv[:, 0], ((0, _JUNK),)),), None,
                          src3, dst3, n, "gcn", nch)
    z = _combine_gcn(acc4[:, :n, :], hb4, dinv, b4.reshape(1, -1))
    return z
